# Initial kernel scaffold; baseline (speedup 1.0000x reference)
import jax
import jax.numpy as jnp
from jax.experimental import pallas as pl


def kernel(x, codebook, Wq, bq, Wk, bk, Wv, bv, Wo, bo, W1, b1, W2, b2, g1, be1, g2, be2):
    raise NotImplementedError("write your pallas kernel here")


# speedup vs baseline: 1.1361x; 1.1361x over previous
import functools
import math

import jax
import jax.numpy as jnp
from jax.experimental import pallas as pl
from jax.experimental.pallas import tpu as pltpu

B, N, D, K, L, H, F = 16, 196, 256, 1024, 4, 8, 1024
DH = D // H


def _ln(x, g, b):
    m = jnp.mean(x, axis=-1, keepdims=True)
    v = jnp.var(x, axis=-1, keepdims=True)
    return (x - m) * jax.lax.rsqrt(v + 1e-5) * g + b


def _mm(a, b):
    return jax.lax.dot_general(
        a, b, (((1,), (0,)), ((), ())), preferred_element_type=jnp.float32)


def _mm_t(a, b):
    return jax.lax.dot_general(
        a, b, (((1,), (1,)), ((), ())), preferred_element_type=jnp.float32)


def _tot_kernel(x_ref, cb_ref, wq_ref, bq_ref, wk_ref, bk_ref, wv_ref, bv_ref,
                wo_ref, bo_ref, w1_ref, b1_ref, w2_ref, b2_ref,
                g1_ref, be1_ref, g2_ref, be2_ref,
                enc_ref, loss_ref):
    xb = x_ref[0]
    cb = cb_ref[...]

    x2 = jnp.sum(xb * xb, axis=1, keepdims=True)
    c2 = jnp.sum(cb * cb, axis=1)
    scores = _mm_t(xb, cb)
    d2 = x2 + c2[None, :] - 2.0 * scores
    d2 = jnp.maximum(d2, 0.0)
    idx = jnp.argmin(d2, axis=1)
    onehot = (jax.lax.broadcasted_iota(jnp.int32, (N, K), 1)
              == idx[:, None]).astype(jnp.float32)
    tok = _mm(onehot, cb)
    loss_ref[0, 0, :] = jnp.full((128,), jnp.sum((tok - xb) ** 2),
                                 dtype=jnp.float32)

    h = tok
    inv_sqrt_dh = 1.0 / math.sqrt(DH)
    for i in range(L):
        q = _mm(h, wq_ref[i]) + bq_ref[i]
        k = _mm(h, wk_ref[i]) + bk_ref[i]
        v = _mm(h, wv_ref[i]) + bv_ref[i]
        outs = []
        for j in range(H):
            sl = slice(j * DH, (j + 1) * DH)
            s = _mm_t(q[:, sl], k[:, sl]) * inv_sqrt_dh
            a = jax.nn.softmax(s, axis=-1)
            outs.append(_mm(a, v[:, sl]))
        o = jnp.concatenate(outs, axis=1)
        o = _mm(o, wo_ref[i]) + bo_ref[i]
        h = _ln(h + o, g1_ref[i], be1_ref[i])
        f = jnp.maximum(_mm(h, w1_ref[i]) + b1_ref[i], 0.0)
        f = _mm(f, w2_ref[i]) + b2_ref[i]
        h = _ln(h + f, g2_ref[i], be2_ref[i])
    enc_ref[0] = h


@jax.jit
def kernel(x, codebook, Wq, bq, Wk, bk, Wv, bv, Wo, bo,
           W1, b1, W2, b2, g1, be1, g2, be2):
    full = lambda s: pl.BlockSpec(s, lambda b: (0,) * len(s))
    in_specs = [
        pl.BlockSpec((1, N, D), lambda b: (b, 0, 0)),
        full((K, D)),
        full((L, D, D)), full((L, D)),
        full((L, D, D)), full((L, D)),
        full((L, D, D)), full((L, D)),
        full((L, D, D)), full((L, D)),
        full((L, D, F)), full((L, F)),
        full((L, F, D)), full((L, D)),
        full((L, D)), full((L, D)),
        full((L, D)), full((L, D)),
    ]
    out_specs = [
        pl.BlockSpec((1, N, D), lambda b: (b, 0, 0)),
        pl.BlockSpec((1, 1, 128), lambda b: (b, 0, 0)),
    ]
    enc, loss_part = pl.pallas_call(
        _tot_kernel,
        grid=(B,),
        in_specs=in_specs,
        out_specs=out_specs,
        out_shape=[
            jax.ShapeDtypeStruct((B, N, D), jnp.float32),
            jax.ShapeDtypeStruct((B, 1, 128), jnp.float32),
        ],
        compiler_params=pltpu.CompilerParams(
            dimension_semantics=("arbitrary",),
        ),
    )(x, codebook, Wq, bq, Wk, bk, Wv, bv, Wo, bo,
      W1, b1, W2, b2, g1, be1, g2, be2)
    loss = jnp.sum(loss_part[:, 0, 0]) / (B * N * D)
    return enc, loss


# speedup vs baseline: 1.2404x; 1.0918x over previous
import math

import jax
import jax.numpy as jnp
from jax.experimental import pallas as pl
from jax.experimental.pallas import tpu as pltpu

B, N, D, K, L, H, F = 16, 196, 256, 1024, 4, 8, 1024
DH = D // H


def _ln(x, g, b):
    m = jnp.mean(x, axis=-1, keepdims=True)
    v = jnp.var(x, axis=-1, keepdims=True)
    return (x - m) * jax.lax.rsqrt(v + 1e-5) * g + b


def _mm(a, b):
    return jax.lax.dot_general(
        a.astype(jnp.bfloat16), b, (((1,), (0,)), ((), ())),
        preferred_element_type=jnp.float32)


def _mm_t(a, b):
    return jax.lax.dot_general(
        a.astype(jnp.bfloat16), b.astype(jnp.bfloat16),
        (((1,), (1,)), ((), ())), preferred_element_type=jnp.float32)


def _split(a):
    hi = a.astype(jnp.bfloat16)
    lo = (a - hi.astype(jnp.float32)).astype(jnp.bfloat16)
    return hi, lo


def _tot_kernel(x_ref, cb_ref, wq_ref, bq_ref, wk_ref, bk_ref, wv_ref, bv_ref,
                wo_ref, bo_ref, w1_ref, b1_ref, w2_ref, b2_ref,
                g1_ref, be1_ref, g2_ref, be2_ref,
                enc_ref, loss_ref):
    xb = x_ref[0]
    cb = cb_ref[...]

    x2 = jnp.sum(xb * xb, axis=1, keepdims=True)
    c2 = jnp.sum(cb * cb, axis=1)
    xh, xl = _split(xb)
    cbh, cbl = _split(cb)
    mmt = lambda a, b: jax.lax.dot_general(
        a, b, (((1,), (1,)), ((), ())), preferred_element_type=jnp.float32)
    scores = mmt(xh, cbh) + mmt(xh, cbl) + mmt(xl, cbh)
    d2 = x2 + c2[None, :] - 2.0 * scores
    d2 = jnp.maximum(d2, 0.0)
    idx = jnp.argmin(d2, axis=1)
    onehot = (jax.lax.broadcasted_iota(jnp.int32, (N, K), 1)
              == idx[:, None]).astype(jnp.bfloat16)
    mmo = lambda a, b: jax.lax.dot_general(
        a, b, (((1,), (0,)), ((), ())), preferred_element_type=jnp.float32)
    tok = mmo(onehot, cbh) + mmo(onehot, cbl)
    loss_ref[0, 0, :] = jnp.full((128,), jnp.sum((tok - xb) ** 2),
                                 dtype=jnp.float32)

    h = tok
    inv_sqrt_dh = 1.0 / math.sqrt(DH)
    for i in range(L):
        q = _mm(h, wq_ref[i]) + bq_ref[i]
        k = _mm(h, wk_ref[i]) + bk_ref[i]
        v = _mm(h, wv_ref[i]) + bv_ref[i]
        outs = []
        for j in range(H):
            sl = slice(j * DH, (j + 1) * DH)
            s = _mm_t(q[:, sl], k[:, sl]) * inv_sqrt_dh
            a = jax.nn.softmax(s, axis=-1)
            outs.append(_mm(a, v[:, sl].astype(jnp.bfloat16)))
        o = jnp.concatenate(outs, axis=1)
        o = _mm(o, wo_ref[i]) + bo_ref[i]
        h = _ln(h + o, g1_ref[i], be1_ref[i])
        f = jnp.maximum(_mm(h, w1_ref[i]) + b1_ref[i], 0.0)
        f = _mm(f, w2_ref[i]) + b2_ref[i]
        h = _ln(h + f, g2_ref[i], be2_ref[i])
    enc_ref[0] = h


@jax.jit
def kernel(x, codebook, Wq, bq, Wk, bk, Wv, bv, Wo, bo,
           W1, b1, W2, b2, g1, be1, g2, be2):
    full = lambda s: pl.BlockSpec(s, lambda b: (0,) * len(s))
    in_specs = [
        pl.BlockSpec((1, N, D), lambda b: (b, 0, 0)),
        full((K, D)),
        full((L, D, D)), full((L, D)),
        full((L, D, D)), full((L, D)),
        full((L, D, D)), full((L, D)),
        full((L, D, D)), full((L, D)),
        full((L, D, F)), full((L, F)),
        full((L, F, D)), full((L, D)),
        full((L, D)), full((L, D)),
        full((L, D)), full((L, D)),
    ]
    out_specs = [
        pl.BlockSpec((1, N, D), lambda b: (b, 0, 0)),
        pl.BlockSpec((1, 1, 128), lambda b: (b, 0, 0)),
    ]
    bf = jnp.bfloat16
    enc, loss_part = pl.pallas_call(
        _tot_kernel,
        grid=(B,),
        in_specs=in_specs,
        out_specs=out_specs,
        out_shape=[
            jax.ShapeDtypeStruct((B, N, D), jnp.float32),
            jax.ShapeDtypeStruct((B, 1, 128), jnp.float32),
        ],
        compiler_params=pltpu.CompilerParams(
            dimension_semantics=("arbitrary",),
        ),
    )(x, codebook, Wq.astype(bf), bq, Wk.astype(bf), bk,
      Wv.astype(bf), bv, Wo.astype(bf), bo,
      W1.astype(bf), b1, W2.astype(bf), b2, g1, be1, g2, be2)
    loss = jnp.sum(loss_part[:, 0, 0]) / (B * N * D)
    return enc, loss


# speedup vs baseline: 1.3283x; 1.0709x over previous
import math

import jax
import jax.numpy as jnp
from jax.experimental import pallas as pl
from jax.experimental.pallas import tpu as pltpu

B, N, D, K, L, H, F = 16, 196, 256, 1024, 4, 8, 1024
DH = D // H


def _ln(x, g, b):
    m = jnp.mean(x, axis=-1, keepdims=True)
    v = jnp.var(x, axis=-1, keepdims=True)
    return (x - m) * jax.lax.rsqrt(v + 1e-5) * g + b


def _mm(a, b):
    return jax.lax.dot_general(
        a.astype(jnp.bfloat16), b, (((1,), (0,)), ((), ())),
        preferred_element_type=jnp.float32)


def _mm_t(a, b):
    return jax.lax.dot_general(
        a.astype(jnp.bfloat16), b.astype(jnp.bfloat16),
        (((1,), (1,)), ((), ())), preferred_element_type=jnp.float32)


def _split(a):
    hi = a.astype(jnp.bfloat16)
    lo = (a - hi.astype(jnp.float32)).astype(jnp.bfloat16)
    return hi, lo


def _tot_kernel(x_ref, cb_ref, wq_ref, bq_ref, wk_ref, bk_ref, wv_ref, bv_ref,
                wo_ref, bo_ref, w1_ref, b1_ref, w2_ref, b2_ref,
                g1_ref, be1_ref, g2_ref, be2_ref,
                enc_ref, loss_ref):
    xb = x_ref[0]
    cb = cb_ref[...]

    x2 = jnp.sum(xb * xb, axis=1, keepdims=True)
    c2 = jnp.sum(cb * cb, axis=1)
    cbh, cbl = _split(cb)
    scores = jax.lax.dot_general(
        xb, cb, (((1,), (1,)), ((), ())),
        precision=jax.lax.Precision.HIGHEST,
        preferred_element_type=jnp.float32)
    d2 = x2 + c2[None, :] - 2.0 * scores
    d2 = jnp.maximum(d2, 0.0)
    idx = jnp.argmin(d2, axis=1)
    onehot = (jax.lax.broadcasted_iota(jnp.int32, (N, K), 1)
              == idx[:, None]).astype(jnp.bfloat16)
    mmo = lambda a, b: jax.lax.dot_general(
        a, b, (((1,), (0,)), ((), ())), preferred_element_type=jnp.float32)
    tok = mmo(onehot, cbh) + mmo(onehot, cbl)
    loss_ref[0, 0, :] = jnp.full((128,), jnp.sum((tok - xb) ** 2),
                                 dtype=jnp.float32)

    h = tok
    inv_sqrt_dh = 1.0 / math.sqrt(DH)
    for i in range(L):
        q = _mm(h, wq_ref[i]) + bq_ref[i]
        k = _mm(h, wk_ref[i]) + bk_ref[i]
        v = _mm(h, wv_ref[i]) + bv_ref[i]
        outs = []
        for j in range(H):
            sl = slice(j * DH, (j + 1) * DH)
            s = _mm_t(q[:, sl], k[:, sl]) * inv_sqrt_dh
            a = jax.nn.softmax(s, axis=-1)
            outs.append(_mm(a, v[:, sl].astype(jnp.bfloat16)))
        o = jnp.concatenate(outs, axis=1)
        o = _mm(o, wo_ref[i]) + bo_ref[i]
        h = _ln(h + o, g1_ref[i], be1_ref[i])
        f = jnp.maximum(_mm(h, w1_ref[i]) + b1_ref[i], 0.0)
        f = _mm(f, w2_ref[i]) + b2_ref[i]
        h = _ln(h + f, g2_ref[i], be2_ref[i])
    enc_ref[0] = h


@jax.jit
def kernel(x, codebook, Wq, bq, Wk, bk, Wv, bv, Wo, bo,
           W1, b1, W2, b2, g1, be1, g2, be2):
    full = lambda s: pl.BlockSpec(s, lambda b: (0,) * len(s))
    in_specs = [
        pl.BlockSpec((1, N, D), lambda b: (b, 0, 0)),
        full((K, D)),
        full((L, D, D)), full((L, D)),
        full((L, D, D)), full((L, D)),
        full((L, D, D)), full((L, D)),
        full((L, D, D)), full((L, D)),
        full((L, D, F)), full((L, F)),
        full((L, F, D)), full((L, D)),
        full((L, D)), full((L, D)),
        full((L, D)), full((L, D)),
    ]
    out_specs = [
        pl.BlockSpec((1, N, D), lambda b: (b, 0, 0)),
        pl.BlockSpec((1, 1, 128), lambda b: (b, 0, 0)),
    ]
    bf = jnp.bfloat16
    enc, loss_part = pl.pallas_call(
        _tot_kernel,
        grid=(B,),
        in_specs=in_specs,
        out_specs=out_specs,
        out_shape=[
            jax.ShapeDtypeStruct((B, N, D), jnp.float32),
            jax.ShapeDtypeStruct((B, 1, 128), jnp.float32),
        ],
        compiler_params=pltpu.CompilerParams(
            dimension_semantics=("arbitrary",),
        ),
    )(x, codebook, Wq.astype(bf), bq, Wk.astype(bf), bk,
      Wv.astype(bf), bv, Wo.astype(bf), bo,
      W1.astype(bf), b1, W2.astype(bf), b2, g1, be1, g2, be2)
    loss = jnp.sum(loss_part[:, 0, 0]) / (B * N * D)
    return enc, loss
